# baseline (speedup 1.0000x reference)
import dataclasses

import jax
import jax.numpy as jnp
from jax import lax
from jax.experimental import pallas as pl
from jax.experimental.pallas import tpu as pltpu
from jax.experimental.pallas import tpu_sc as plsc

N = 10000
E = 160000
D = 256

NC = 2
NS = 16
NW = NC * NS
OWN = 320
ACC_R = 328
TRASH = OWN
SCE = 1600
NSUP = E // SCE
FIRE = 64
SBUF = 176
BTRASH = 160


def _agg_body(h_hbm, src_hbm, dst_hbm, z_hbm, agg_hbm,
              acc, srcstA, dststA, srcstB, dststB, srcbuf, locbuf, rows_v,
              spst, smem64, sem, semA, semB, sem0, sem1):
    c = lax.axis_index("c")
    s = lax.axis_index("s")
    wid = s * NC + c
    base = wid * OWN

    pltpu.sync_copy(z_hbm, acc)

    def _bounce_locs(bo, cnt, reg):
        so = s * 2 * FIRE + reg * FIRE
        pltpu.sync_copy(locbuf.at[pl.ds(bo, cnt)],
                        spst.at[pl.ds(so, cnt)])
        pltpu.sync_copy(spst.at[pl.ds(so, cnt)],
                        smem64.at[pl.ds(reg * FIRE, cnt)])

    def _accum_rows(nrows, reg):
        @pl.loop(0, nrows, step=2)
        def _row(r):
            ro0 = smem64[reg * FIRE + r] * D
            ro1 = smem64[reg * FIRE + r + 1] * D
            for j in range(0, D, 16):
                plsc.addupdate(acc.at[pl.ds(ro0 + j, 16)],
                               rows_v[reg * FIRE + r, pl.ds(j, 16)])
                plsc.addupdate(acc.at[pl.ds(ro1 + j, 16)],
                               rows_v[reg * FIRE + r + 1, pl.ds(j, 16)])

    def _scan_step(i, carry, srcst, dstst):
        fv, pend = carry
        for u in range(4):
            o = i * 64 + u * 16
            d = dstst[pl.ds(o, 16)]
            sv = srcst[pl.ds(o, 16)]
            m = (d >= base) & (d < base + OWN)
            pos = plsc.cumsum(m.astype(jnp.int32))
            idx = jnp.where(m, fv + pos - 1, BTRASH)
            plsc.store_scatter(srcbuf, [idx], sv)
            plsc.store_scatter(locbuf, [idx], d - base)
            fv = fv + plsc.all_reduce_population_count(m)

        def _move_left():
            for u in range(4):
                st = srcbuf[pl.ds(FIRE + u * 16, 16)]
                lt = locbuf[pl.ds(FIRE + u * 16, 16)]
                srcbuf[pl.ds(u * 16, 16)] = st
                locbuf[pl.ds(u * 16, 16)] = lt

        def _fire_into(reg, gsem, carry):
            f, pend = carry
            _bounce_locs(0, FIRE, reg)
            pltpu.make_async_copy(
                h_hbm.at[srcbuf.at[pl.ds(0, FIRE)]],
                rows_v.at[pl.ds(reg * FIRE, FIRE)], gsem).start()
            _move_left()
            return f - FIRE, pend

        def _fire_r0(carry):
            f, pend = _fire_into(0, sem0, carry)

            @pl.when(pend == 2)
            def _drain_r1():
                pltpu.make_async_copy(
                    h_hbm.at[srcbuf.at[pl.ds(0, FIRE)]],
                    rows_v.at[pl.ds(FIRE, FIRE)], sem1).wait()
                _accum_rows(FIRE, 1)

            return f, jnp.int32(1)

        def _fire_r1(carry):
            f, pend = _fire_into(1, sem1, carry)
            pltpu.make_async_copy(
                h_hbm.at[srcbuf.at[pl.ds(0, FIRE)]],
                rows_v.at[pl.ds(0, FIRE)], sem0).wait()
            _accum_rows(FIRE, 0)
            return f, jnp.int32(2)

        def _fire(carry):
            f, pend = carry
            return lax.cond(pend == 1, _fire_r1, _fire_r0, carry)

        return lax.cond(jnp.any(fv >= FIRE), _fire, lambda cr: cr,
                        (fv, pend))

    def _stage_start(sp, sbuf, dbuf, ssem):
        off = sp * SCE
        pltpu.make_async_copy(src_hbm.at[pl.ds(off, SCE)], sbuf, ssem).start()
        pltpu.make_async_copy(dst_hbm.at[pl.ds(off, SCE)], dbuf, ssem).start()

    def _stage_wait(sbuf, dbuf, ssem):
        pltpu.make_async_copy(src_hbm.at[pl.ds(0, SCE)], sbuf, ssem).wait()
        pltpu.make_async_copy(dst_hbm.at[pl.ds(0, SCE)], dbuf, ssem).wait()

    def _scan_chunk(carry, sbuf, dbuf):
        def _step(i, cr):
            return _scan_step(i, cr, sbuf, dbuf)
        return lax.fori_loop(0, SCE // 64, _step, carry)

    _stage_start(0, srcstA, dststA, semA)

    def _super2(k, carry):
        _stage_start(2 * k + 1, srcstB, dststB, semB)
        _stage_wait(srcstA, dststA, semA)
        carry = _scan_chunk(carry, srcstA, dststA)

        @pl.when(k < NSUP // 2 - 1)
        def _prefetch_next():
            _stage_start(2 * k + 2, srcstA, dststA, semA)

        _stage_wait(srcstB, dststB, semB)
        return _scan_chunk(carry, srcstB, dststB)

    fill_v, pend = lax.fori_loop(
        0, NSUP // 2, _super2,
        (jnp.zeros((16,), jnp.int32), jnp.int32(0)))
    fill = jnp.max(fill_v)

    @pl.when(pend == 1)
    def _drain0():
        pltpu.make_async_copy(h_hbm.at[srcbuf.at[pl.ds(0, FIRE)]],
                              rows_v.at[pl.ds(0, FIRE)], sem0).wait()
        _accum_rows(FIRE, 0)

    @pl.when(pend == 2)
    def _drain1():
        pltpu.make_async_copy(h_hbm.at[srcbuf.at[pl.ds(0, FIRE)]],
                              rows_v.at[pl.ds(FIRE, FIRE)], sem1).wait()
        _accum_rows(FIRE, 1)

    pad_idx = fill + lax.iota(jnp.int32, 16)
    plsc.store_scatter(srcbuf, [pad_idx], lax.iota(jnp.int32, 16) * 8)
    plsc.store_scatter(locbuf, [pad_idx], jnp.full((16,), TRASH, jnp.int32))
    nbat = (fill + 15) // 16

    @pl.loop(0, nbat)
    def _tail(b):
        pltpu.async_copy(h_hbm.at[srcbuf.at[pl.ds(b * 16, 16)]],
                         rows_v.at[pl.ds(0, 16)], sem).wait()
        _bounce_locs(b * 16, 16, 0)
        _accum_rows(16, 0)

    @pl.when(wid < NW - 1)
    def _wb():
        pltpu.sync_copy(acc.at[pl.ds(0, OWN * D)],
                        agg_hbm.at[pl.ds(base * D, OWN * D)])

    @pl.when(wid == NW - 1)
    def _wb_last():
        pltpu.sync_copy(acc.at[pl.ds(0, (N - (NW - 1) * OWN) * D)],
                        agg_hbm.at[pl.ds(base * D, (N - (NW - 1) * OWN) * D)])


def _transpose_body(f_ref, h_ref):
    h_ref[...] = f_ref[...].T


def _linear_body(a_ref, w_ref, b_ref, o_ref):
    yt = lax.dot_general(w_ref[...], a_ref[...], (((1,), (1,)), ((), ())),
                         preferred_element_type=jnp.float32,
                         precision=lax.Precision.HIGHEST)
    o_ref[...] = jnp.maximum(yt + b_ref[...], 0.0)


def kernel(feature, edge_index, W, b):
    src = edge_index[0]
    dst = edge_index[1]
    zeros = jnp.zeros((ACC_R * D,), jnp.float32)

    BT = 512
    h = pl.pallas_call(
        _transpose_body,
        grid=(pl.cdiv(N, BT),),
        in_specs=[pl.BlockSpec((D, BT), lambda i: (0, i))],
        out_specs=pl.BlockSpec((BT, D), lambda i: (i, 0)),
        out_shape=jax.ShapeDtypeStruct((N, D), jnp.float32),
    )(feature)

    cp = pltpu.CompilerParams()
    if "needs_layout_passes" in pltpu.CompilerParams.__dataclass_fields__:
        cp = dataclasses.replace(cp, needs_layout_passes=False)
    agg1d = pl.kernel(
        _agg_body,
        out_type=jax.ShapeDtypeStruct((N * D,), jnp.float32),
        compiler_params=cp,
        mesh=plsc.VectorSubcoreMesh(core_axis_name="c", subcore_axis_name="s"),
        scratch_types=[
            pltpu.VMEM((ACC_R * D,), jnp.float32),
            pltpu.VMEM((SCE,), jnp.int32),
            pltpu.VMEM((SCE,), jnp.int32),
            pltpu.VMEM((SCE,), jnp.int32),
            pltpu.VMEM((SCE,), jnp.int32),
            pltpu.VMEM((SBUF,), jnp.int32),
            pltpu.VMEM((SBUF,), jnp.int32),
            pltpu.VMEM((2 * FIRE, D), jnp.float32),
            pltpu.VMEM_SHARED((NS * 2 * FIRE,), jnp.int32),
            pltpu.SMEM((2 * FIRE,), jnp.int32),
            pltpu.SemaphoreType.DMA,
            pltpu.SemaphoreType.DMA,
            pltpu.SemaphoreType.DMA,
            pltpu.SemaphoreType.DMA,
            pltpu.SemaphoreType.DMA,
        ],
    )(h, src, dst, zeros)
    agg = agg1d.reshape(N, D)

    BN = 512
    out = pl.pallas_call(
        _linear_body,
        grid=(pl.cdiv(N, BN),),
        in_specs=[pl.BlockSpec((BN, D), lambda i: (i, 0)),
                  pl.BlockSpec((D, D), lambda i: (0, 0)),
                  pl.BlockSpec((D, 1), lambda i: (0, 0))],
        out_specs=pl.BlockSpec((D, BN), lambda i: (0, i)),
        out_shape=jax.ShapeDtypeStruct((D, N), jnp.float32),
    )(agg, W, b.reshape(D, 1))

    return out


# speedup vs baseline: 1.0047x; 1.0047x over previous
import dataclasses

import jax
import jax.numpy as jnp
from jax import lax
from jax.experimental import pallas as pl
from jax.experimental.pallas import tpu as pltpu
from jax.experimental.pallas import tpu_sc as plsc

N = 10000
E = 160000
D = 256

NC = 2
NS = 16
NW = NC * NS
OWN = 320
ACC_R = 328
TRASH = OWN
SCE = 1600
NSUP = E // SCE
FIRE = 64
SBUF = 176
BTRASH = 160


def _agg_body(h_hbm, src_hbm, dst_hbm, z_hbm, agg_hbm,
              acc, srcstA, dststA, srcstB, dststB, srcbuf, locbuf, rows_v,
              spst, smem64, sem, semA, semB, sem0, sem1):
    c = lax.axis_index("c")
    s = lax.axis_index("s")
    wid = s * NC + c
    base = wid * OWN

    pltpu.sync_copy(z_hbm, acc)

    def _bounce_locs(bo, cnt, reg):
        so = s * 2 * FIRE + reg * FIRE
        pltpu.sync_copy(locbuf.at[pl.ds(bo, cnt)],
                        spst.at[pl.ds(so, cnt)])
        pltpu.sync_copy(spst.at[pl.ds(so, cnt)],
                        smem64.at[pl.ds(reg * FIRE, cnt)])

    def _accum_rows(nrows, reg):
        @pl.loop(0, nrows)
        def _row(r):
            rowoff = smem64[reg * FIRE + r] * D
            for j in range(0, D, 16):
                plsc.addupdate(acc.at[pl.ds(rowoff + j, 16)],
                               rows_v[reg * FIRE + r, pl.ds(j, 16)])

    def _scan_step(i, carry, srcst, dstst):
        fv, pend = carry
        for u in range(4):
            o = i * 64 + u * 16
            d = dstst[pl.ds(o, 16)]
            sv = srcst[pl.ds(o, 16)]
            m = (d >= base) & (d < base + OWN)
            pos = plsc.cumsum(m.astype(jnp.int32))
            idx = jnp.where(m, fv + pos - 1, BTRASH)
            plsc.store_scatter(srcbuf, [idx], sv)
            plsc.store_scatter(locbuf, [idx], d - base)
            fv = fv + plsc.all_reduce_population_count(m)

        def _move_left():
            for u in range(4):
                st = srcbuf[pl.ds(FIRE + u * 16, 16)]
                lt = locbuf[pl.ds(FIRE + u * 16, 16)]
                srcbuf[pl.ds(u * 16, 16)] = st
                locbuf[pl.ds(u * 16, 16)] = lt

        def _fire_into(reg, gsem, carry):
            f, pend = carry
            _bounce_locs(0, FIRE, reg)
            pltpu.make_async_copy(
                h_hbm.at[srcbuf.at[pl.ds(0, FIRE)]],
                rows_v.at[pl.ds(reg * FIRE, FIRE)], gsem).start()
            _move_left()
            return f - FIRE, pend

        def _fire_r0(carry):
            f, pend = _fire_into(0, sem0, carry)

            @pl.when(pend == 2)
            def _drain_r1():
                pltpu.make_async_copy(
                    h_hbm.at[srcbuf.at[pl.ds(0, FIRE)]],
                    rows_v.at[pl.ds(FIRE, FIRE)], sem1).wait()
                _accum_rows(FIRE, 1)

            return f, jnp.int32(1)

        def _fire_r1(carry):
            f, pend = _fire_into(1, sem1, carry)
            pltpu.make_async_copy(
                h_hbm.at[srcbuf.at[pl.ds(0, FIRE)]],
                rows_v.at[pl.ds(0, FIRE)], sem0).wait()
            _accum_rows(FIRE, 0)
            return f, jnp.int32(2)

        def _fire(carry):
            f, pend = carry
            return lax.cond(pend == 1, _fire_r1, _fire_r0, carry)

        return lax.cond(jnp.any(fv >= FIRE), _fire, lambda cr: cr,
                        (fv, pend))

    def _stage_start(sp, sbuf, dbuf, ssem):
        off = sp * SCE
        pltpu.make_async_copy(src_hbm.at[pl.ds(off, SCE)], sbuf, ssem).start()
        pltpu.make_async_copy(dst_hbm.at[pl.ds(off, SCE)], dbuf, ssem).start()

    def _stage_wait(sbuf, dbuf, ssem):
        pltpu.make_async_copy(src_hbm.at[pl.ds(0, SCE)], sbuf, ssem).wait()
        pltpu.make_async_copy(dst_hbm.at[pl.ds(0, SCE)], dbuf, ssem).wait()

    def _scan_chunk(carry, sbuf, dbuf):
        def _step(i, cr):
            return _scan_step(i, cr, sbuf, dbuf)
        return lax.fori_loop(0, SCE // 64, _step, carry)

    _stage_start(0, srcstA, dststA, semA)

    def _super2(k, carry):
        _stage_start(2 * k + 1, srcstB, dststB, semB)
        _stage_wait(srcstA, dststA, semA)
        carry = _scan_chunk(carry, srcstA, dststA)

        @pl.when(k < NSUP // 2 - 1)
        def _prefetch_next():
            _stage_start(2 * k + 2, srcstA, dststA, semA)

        _stage_wait(srcstB, dststB, semB)
        return _scan_chunk(carry, srcstB, dststB)

    fill_v, pend = lax.fori_loop(
        0, NSUP // 2, _super2,
        (jnp.zeros((16,), jnp.int32), jnp.int32(0)))
    fill = jnp.max(fill_v)

    @pl.when(pend == 1)
    def _drain0():
        pltpu.make_async_copy(h_hbm.at[srcbuf.at[pl.ds(0, FIRE)]],
                              rows_v.at[pl.ds(0, FIRE)], sem0).wait()
        _accum_rows(FIRE, 0)

    @pl.when(pend == 2)
    def _drain1():
        pltpu.make_async_copy(h_hbm.at[srcbuf.at[pl.ds(0, FIRE)]],
                              rows_v.at[pl.ds(FIRE, FIRE)], sem1).wait()
        _accum_rows(FIRE, 1)

    pad_idx = fill + lax.iota(jnp.int32, 16)
    plsc.store_scatter(srcbuf, [pad_idx], lax.iota(jnp.int32, 16) * 8)
    plsc.store_scatter(locbuf, [pad_idx], jnp.full((16,), TRASH, jnp.int32))
    nbat = (fill + 15) // 16

    @pl.loop(0, nbat)
    def _tail(b):
        pltpu.async_copy(h_hbm.at[srcbuf.at[pl.ds(b * 16, 16)]],
                         rows_v.at[pl.ds(0, 16)], sem).wait()
        _bounce_locs(b * 16, 16, 0)
        _accum_rows(16, 0)

    @pl.when(wid < NW - 1)
    def _wb():
        pltpu.sync_copy(acc.at[pl.ds(0, OWN * D)],
                        agg_hbm.at[pl.ds(base * D, OWN * D)])

    @pl.when(wid == NW - 1)
    def _wb_last():
        pltpu.sync_copy(acc.at[pl.ds(0, (N - (NW - 1) * OWN) * D)],
                        agg_hbm.at[pl.ds(base * D, (N - (NW - 1) * OWN) * D)])


def _transpose_body(f_ref, h_ref):
    h_ref[...] = f_ref[...].T


def _linear_body(a_ref, w_ref, b_ref, o_ref):
    dn = (((1,), (1,)), ((), ()))
    a = a_ref[...]
    w = w_ref[...]
    a_hi = a.astype(jnp.bfloat16).astype(jnp.float32)
    w_hi = w.astype(jnp.bfloat16).astype(jnp.float32)
    a_lo = a - a_hi
    w_lo = w - w_hi
    yt = (lax.dot_general(w_hi, a_hi, dn, preferred_element_type=jnp.float32)
          + lax.dot_general(w_hi, a_lo, dn, preferred_element_type=jnp.float32)
          + lax.dot_general(w_lo, a_hi, dn, preferred_element_type=jnp.float32))
    o_ref[...] = jnp.maximum(yt + b_ref[...], 0.0)


def kernel(feature, edge_index, W, b):
    src = edge_index[0]
    dst = edge_index[1]
    zeros = jnp.zeros((ACC_R * D,), jnp.float32)

    BT = 512
    h = pl.pallas_call(
        _transpose_body,
        grid=(pl.cdiv(N, BT),),
        in_specs=[pl.BlockSpec((D, BT), lambda i: (0, i))],
        out_specs=pl.BlockSpec((BT, D), lambda i: (i, 0)),
        out_shape=jax.ShapeDtypeStruct((N, D), jnp.float32),
    )(feature)

    cp = pltpu.CompilerParams()
    if "needs_layout_passes" in pltpu.CompilerParams.__dataclass_fields__:
        cp = dataclasses.replace(cp, needs_layout_passes=False)
    agg1d = pl.kernel(
        _agg_body,
        out_type=jax.ShapeDtypeStruct((N * D,), jnp.float32),
        compiler_params=cp,
        mesh=plsc.VectorSubcoreMesh(core_axis_name="c", subcore_axis_name="s"),
        scratch_types=[
            pltpu.VMEM((ACC_R * D,), jnp.float32),
            pltpu.VMEM((SCE,), jnp.int32),
            pltpu.VMEM((SCE,), jnp.int32),
            pltpu.VMEM((SCE,), jnp.int32),
            pltpu.VMEM((SCE,), jnp.int32),
            pltpu.VMEM((SBUF,), jnp.int32),
            pltpu.VMEM((SBUF,), jnp.int32),
            pltpu.VMEM((2 * FIRE, D), jnp.float32),
            pltpu.VMEM_SHARED((NS * 2 * FIRE,), jnp.int32),
            pltpu.SMEM((2 * FIRE,), jnp.int32),
            pltpu.SemaphoreType.DMA,
            pltpu.SemaphoreType.DMA,
            pltpu.SemaphoreType.DMA,
            pltpu.SemaphoreType.DMA,
            pltpu.SemaphoreType.DMA,
        ],
    )(h, src, dst, zeros)
    agg = agg1d.reshape(N, D)

    BN = 512
    out = pl.pallas_call(
        _linear_body,
        grid=(pl.cdiv(N, BN),),
        in_specs=[pl.BlockSpec((BN, D), lambda i: (i, 0)),
                  pl.BlockSpec((D, D), lambda i: (0, 0)),
                  pl.BlockSpec((D, 1), lambda i: (0, 0))],
        out_specs=pl.BlockSpec((D, BN), lambda i: (0, i)),
        out_shape=jax.ShapeDtypeStruct((D, N), jnp.float32),
    )(agg, W, b.reshape(D, 1))

    return out
